# Initial kernel scaffold; baseline (speedup 1.0000x reference)
import jax
import jax.numpy as jnp
from jax.experimental import pallas as pl


def kernel(x, edge_index, edge_attr, batch, W1_0, b1_0, W2_0, b2_0, bn_g_0, bn_b_0, W1_1, b1_1, W2_1, b2_1, bn_g_1, bn_b_1):
    raise NotImplementedError("write your pallas kernel here")


# speedup vs baseline: 2.9280x; 2.9280x over previous
import functools

import jax
import jax.numpy as jnp
from jax import lax
from jax.experimental import pallas as pl
from jax.experimental.pallas import tpu as pltpu
from jax.experimental.pallas import tpu_sc as plsc

N = 10000
E = 320000
D = 128

NC = 2
NS = 16
CK = 128
CH = 80
EPW = CH * CK
E_PAD = NC * NS * EPW
NP = 10112
RPT = NP // NS


def _sc_agg(h, src_p, dst_p, zinit):
    mesh = plsc.VectorSubcoreMesh(core_axis_name="c", subcore_axis_name="s")

    @functools.partial(
        pl.kernel,
        mesh=mesh,
        out_type=jax.ShapeDtypeStruct((NC, NP, D), jnp.float32),
        scratch_types=[
            pltpu.VMEM((CH, CK), jnp.int32),
            pltpu.VMEM((CH, CK), jnp.int32),
            pltpu.VMEM((CK, D), jnp.float32),
            pltpu.VMEM_SHARED((NP, D), jnp.float32),
            pltpu.SemaphoreType.DMA,
        ],
    )
    def k(h_hbm, src_hbm, dst_hbm, z_hbm, out_hbm, src_v, dst_v, rows_v, agg_sh, gsem):
        c = lax.axis_index("c")
        s = lax.axis_index("s")
        pltpu.sync_copy(src_hbm.at[c, s], src_v)
        pltpu.sync_copy(dst_hbm.at[c, s], dst_v)
        pltpu.sync_copy(z_hbm.at[pl.ds(s * RPT, RPT)], agg_sh.at[pl.ds(s * RPT, RPT)])
        plsc.subcore_barrier()

        @pl.loop(0, CH)
        def _(j):
            pltpu.async_copy(h_hbm.at[src_v.at[j]], rows_v, gsem).wait()
            pltpu.sync_copy(rows_v, agg_sh.at[dst_v.at[j]], add=True)

        plsc.subcore_barrier()
        pltpu.sync_copy(agg_sh.at[pl.ds(s * RPT, RPT)],
                        out_hbm.at[c, pl.ds(s * RPT, RPT)])

    return k(h, src_p, dst_p, zinit)


def _tc_mlp_bn(h, a0, a1, W1, b1, W2, b2, g, bb, relu_out):
    def body(h_ref, a0_ref, a1_ref, w1_ref, b1_ref, w2_ref, b2_ref, g_ref,
             bb_ref, o_ref):
        z = h_ref[...] + a0_ref[...] + a1_ref[...]
        t = jnp.dot(z, w1_ref[...], preferred_element_type=jnp.float32) + b1_ref[...]
        t = jnp.maximum(t, 0.0)
        u = jnp.dot(t, w2_ref[...], preferred_element_type=jnp.float32) + b2_ref[...]
        mu = jnp.mean(u, axis=0, keepdims=True)
        var = jnp.mean(jnp.square(u - mu), axis=0, keepdims=True)
        o = g_ref[...] * (u - mu) * lax.rsqrt(var + 1e-5) + bb_ref[...]
        if relu_out:
            o = jnp.maximum(o, 0.0)
        o_ref[...] = o

    return pl.pallas_call(
        body,
        out_shape=jax.ShapeDtypeStruct((N, D), jnp.float32),
    )(h, a0, a1, W1, b1, W2, b2, g, bb)


def kernel(x, edge_index, edge_attr, batch,
           W1_0, b1_0, W2_0, b2_0, bn_g_0, bn_b_0,
           W1_1, b1_1, W2_1, b2_1, bn_g_1, bn_b_1):
    x = x.astype(jnp.float32)
    pad = E_PAD - E
    src = jnp.concatenate([edge_index[0], jnp.zeros((pad,), jnp.int32)])
    dst = jnp.concatenate([edge_index[1], jnp.full((pad,), N, jnp.int32)])
    src_p = src.reshape(NC, NS, CH, CK)
    dst_p = dst.reshape(NC, NS, CH, CK)
    zinit = jnp.zeros((NP, D), jnp.float32)

    params = [
        (W1_0, b1_0, W2_0, b2_0, bn_g_0, bn_b_0),
        (W1_1, b1_1, W2_1, b2_1, bn_g_1, bn_b_1),
    ]
    h = x
    for layer, (W1, b1, W2, b2, g, bb) in enumerate(params):
        parts = _sc_agg(h, src_p, dst_p, zinit)
        h = _tc_mlp_bn(h, parts[0, :N], parts[1, :N], W1,
                       b1.reshape(1, D), W2, b2.reshape(1, D),
                       g.reshape(1, D), bb.reshape(1, D),
                       relu_out=(layer == 0))
    return h


# speedup vs baseline: 3.2576x; 1.1126x over previous
import functools

import jax
import jax.numpy as jnp
from jax import lax
from jax.experimental import pallas as pl
from jax.experimental.pallas import tpu as pltpu
from jax.experimental.pallas import tpu_sc as plsc

N = 10000
E = 320000
D = 128

NC = 2
NS = 16
CK = 128
CH = 80
NH = 2
CH2 = CH // NH
EPW = CH * CK
E_PAD = NC * NS * EPW
NP = 10112
RPT = NP // NS


def _sc_agg(h, src_p, dst_p, zinit):
    mesh = plsc.VectorSubcoreMesh(core_axis_name="c", subcore_axis_name="s")

    @functools.partial(
        pl.kernel,
        mesh=mesh,
        out_type=jax.ShapeDtypeStruct((NC, NP, D), jnp.float32),
        scratch_types=[
            pltpu.VMEM((CH2, CK), jnp.int32),
            pltpu.VMEM((CH2, CK), jnp.int32),
            pltpu.VMEM((2, CK, D), jnp.float32),
            pltpu.VMEM_SHARED((NP, D), jnp.float32),
            pltpu.SemaphoreType.DMA,
        ],
    )
    def k(h_hbm, src_hbm, dst_hbm, z_hbm, out_hbm, src_v, dst_v, rows_v, agg_sh,
          sem0):
        c = lax.axis_index("c")
        s = lax.axis_index("s")
        pltpu.sync_copy(z_hbm.at[pl.ds(s * RPT, RPT)], agg_sh.at[pl.ds(s * RPT, RPT)])
        plsc.subcore_barrier()

        for ph in range(NH):
            pltpu.sync_copy(src_hbm.at[c, s, ph], src_v)
            pltpu.sync_copy(dst_hbm.at[c, s, ph], dst_v)

            @pl.loop(0, CH2 + 1)
            def _(j):
                b = lax.rem(j, 2)

                @pl.when(j < CH2)
                def _():
                    pltpu.async_copy(h_hbm.at[src_v.at[j]], rows_v.at[b], sem0)

                @pl.when(j > 0)
                def _():
                    pb = lax.rem(j + 1, 2)
                    pltpu.make_async_copy(h_hbm.at[pl.ds(0, CK)], rows_v.at[pb],
                                          sem0).wait()
                    pltpu.sync_copy(rows_v.at[pb], agg_sh.at[dst_v.at[j - 1]],
                                    add=True)

        plsc.subcore_barrier()
        pltpu.sync_copy(agg_sh.at[pl.ds(s * RPT, RPT)],
                        out_hbm.at[c, pl.ds(s * RPT, RPT)])

    return k(h, src_p, dst_p, zinit)


def _tc_mlp_bn(h, a0, a1, W1, b1, W2, b2, g, bb, relu_out):
    def body(h_ref, a0_ref, a1_ref, w1_ref, b1_ref, w2_ref, b2_ref, g_ref,
             bb_ref, o_ref):
        z = h_ref[...] + a0_ref[...] + a1_ref[...]
        t = jnp.dot(z, w1_ref[...], preferred_element_type=jnp.float32) + b1_ref[...]
        t = jnp.maximum(t, 0.0)
        u = jnp.dot(t, w2_ref[...], preferred_element_type=jnp.float32) + b2_ref[...]
        mu = jnp.mean(u, axis=0, keepdims=True)
        var = jnp.mean(jnp.square(u - mu), axis=0, keepdims=True)
        o = g_ref[...] * (u - mu) * lax.rsqrt(var + 1e-5) + bb_ref[...]
        if relu_out:
            o = jnp.maximum(o, 0.0)
        o_ref[...] = o

    return pl.pallas_call(
        body,
        out_shape=jax.ShapeDtypeStruct((N, D), jnp.float32),
    )(h, a0, a1, W1, b1, W2, b2, g, bb)


def kernel(x, edge_index, edge_attr, batch,
           W1_0, b1_0, W2_0, b2_0, bn_g_0, bn_b_0,
           W1_1, b1_1, W2_1, b2_1, bn_g_1, bn_b_1):
    x = x.astype(jnp.float32)
    pad = E_PAD - E
    src = jnp.concatenate([edge_index[0], jnp.zeros((pad,), jnp.int32)])
    dst = jnp.concatenate([edge_index[1], jnp.full((pad,), N, jnp.int32)])
    src_p = src.reshape(NC, NS, NH, CH2, CK)
    dst_p = dst.reshape(NC, NS, NH, CH2, CK)
    zinit = jnp.zeros((NP, D), jnp.float32)

    params = [
        (W1_0, b1_0, W2_0, b2_0, bn_g_0, bn_b_0),
        (W1_1, b1_1, W2_1, b2_1, bn_g_1, bn_b_1),
    ]
    h = x
    for layer, (W1, b1, W2, b2, g, bb) in enumerate(params):
        parts = _sc_agg(h, src_p, dst_p, zinit)
        h = _tc_mlp_bn(h, parts[0, :N], parts[1, :N], W1,
                       b1.reshape(1, D), W2, b2.reshape(1, D),
                       g.reshape(1, D), bb.reshape(1, D),
                       relu_out=(layer == 0))
    return h
